# Initial kernel scaffold; baseline (speedup 1.0000x reference)
import jax
import jax.numpy as jnp
from jax.experimental import pallas as pl


def kernel(x_atom, x_bond, x_glob, a2b_src, a2b_dst, b2a_src, b2a_dst, g2a_src, g2a_dst, a2g_src, a2g_dst, g2b_src, g2b_dst, b2g_src, b2g_dst, params):
    raise NotImplementedError("write your pallas kernel here")


# speedup vs baseline: 1.1165x; 1.1165x over previous
import functools

import jax
import jax.numpy as jnp
from jax import lax
from jax.experimental import pallas as pl

_NA, _NB, _NG = 100000, 100000, 5000
_H, _DH = 8, 8
_HID = 64


def _fc_body(v_ref, w0, b0, w1, b1, w2, b2, out_ref):
    v = v_ref[...]
    h = jnp.maximum(jnp.dot(v, w0[...], preferred_element_type=jnp.float32) + b0[...], 0.0)
    h = jnp.maximum(jnp.dot(h, w1[...], preferred_element_type=jnp.float32) + b1[...], 0.0)
    o = jnp.dot(h, w2[...], preferred_element_type=jnp.float32) + b2[...]
    out_ref[...] = o


def _fc_head(v, p):
    n = v.shape[0]
    blk = 2000
    grid = (n // blk,)
    out = pl.pallas_call(
        _fc_body,
        grid=grid,
        in_specs=[
            pl.BlockSpec((blk, v.shape[1]), lambda i: (i, 0)),
            pl.BlockSpec((192, 64), lambda i: (0, 0)),
            pl.BlockSpec((64,), lambda i: (0,)),
            pl.BlockSpec((64, 64), lambda i: (0, 0)),
            pl.BlockSpec((64,), lambda i: (0,)),
            pl.BlockSpec((64, 1), lambda i: (0, 0)),
            pl.BlockSpec((1,), lambda i: (0,)),
        ],
        out_specs=pl.BlockSpec((blk, 1), lambda i: (i, 0)),
        out_shape=jax.ShapeDtypeStruct((n, 1), jnp.float32),
    )(v, p["fc0_W"], p["fc0_b"], p["fc1_W"], p["fc1_b"], p["fc2_W"], p["fc2_b"])
    return out.reshape(-1)


def _gat(zs, zd, src, dst, a_s, a_d, n_dst):
    es = jnp.sum(zs * a_s, axis=-1)
    ed = jnp.sum(zd * a_d, axis=-1)
    mb = jnp.max(es, axis=0) + jnp.max(ed, axis=0)
    e = jax.nn.leaky_relu(es[src] + ed[dst], 0.2)
    ex = jnp.exp(e - mb)
    den = jax.ops.segment_sum(ex, dst, num_segments=n_dst)
    alpha = ex / (den[dst] + 1e-9)
    return jax.ops.segment_sum(alpha[:, :, None] * zs[src], dst, num_segments=n_dst)


def kernel(x_atom, x_bond, x_glob, a2b_src, a2b_dst, b2a_src, b2a_dst, g2a_src, g2a_dst, a2g_src, a2g_dst, g2b_src, g2b_dst, b2g_src, b2g_dst, params):
    p = params
    edges = {"a2b": (a2b_src, a2b_dst), "b2a": (b2a_src, b2a_dst), "g2a": (g2a_src, g2a_dst), "a2g": (a2g_src, a2g_dst), "g2b": (g2b_src, g2b_dst), "b2g": (b2g_src, b2g_dst)}
    h = {"atom": x_atom @ p["U_atom"], "bond": x_bond @ p["U_bond"], "global": x_glob @ p["U_glob"]}
    sizes = {"atom": _NA, "bond": _NB, "global": _NG}
    mech = {"atom": (("bond", "b2a"), ("global", "g2a")), "bond": (("atom", "a2b"), ("global", "g2b")), "global": (("atom", "a2g"), ("bond", "b2g"))}
    for l in range(3):
        for nt in ("atom", "bond", "global"):
            zd = (h[nt] @ p["W%d_%s" % (l, nt)]).reshape(-1, _H, _DH)
            acc = jnp.zeros_like(zd)
            for (snt, et) in mech[nt]:
                zs = (h[snt] @ p["W%d_%s" % (l, snt)]).reshape(-1, _H, _DH)
                src, dst = edges[et]
                acc = acc + _gat(zs, zd, src, dst, p["asrc%d_%s" % (l, et)], p["adst%d_%s" % (l, et)], sizes[nt])
            h[nt] = jax.nn.elu(acc).reshape(-1, _H * _DH)
    src, dst = edges["a2b"]
    gathered = h["atom"][src]
    cnt = jax.ops.segment_sum(jnp.ones((src.shape[0],), jnp.float32), dst, num_segments=_NB)
    mean = jax.ops.segment_sum(gathered, dst, num_segments=_NB) / jnp.maximum(cnt, 1.0)[:, None]
    mx = jax.ops.segment_max(gathered, dst, num_segments=_NB)
    mx = jnp.where(jnp.isfinite(mx), mx, 0.0)
    v = jnp.concatenate([h["bond"], mean, mx], axis=1)
    return _fc_head(v, p)


# speedup vs baseline: 4.5831x; 4.1050x over previous
import functools

import jax
import jax.numpy as jnp
from jax import lax
from jax.experimental import pallas as pl
from jax.experimental.pallas import tpu as pltpu
from jax.experimental.pallas import tpu_sc as plsc

NAT, NBN, NGL = 100000, 100000, 5000
NH, NDH, HID = 8, 8, 64
ROWB = 1000


def _rup(x, m):
    return ((x + m - 1) // m) * m


def _np(n):
    return _rup(n + 8, 128)


NATP, NBNP, NGLP = _np(NAT), _np(NBN), _np(NGL)


def _elu(x):
    return jnp.where(x > 0, x, jnp.exp(x) - 1.0)


def _mm_init_body(x_ref, u_ref, o_ref):
    o_ref[...] = jnp.dot(x_ref[...], u_ref[...], preferred_element_type=jnp.float32)


def _mm_init(x, u):
    n = x.shape[0]
    return pl.pallas_call(
        _mm_init_body,
        grid=(n // ROWB,),
        in_specs=[pl.BlockSpec((ROWB, x.shape[1]), lambda i: (i, 0)),
                  pl.BlockSpec(u.shape, lambda i: (0, 0))],
        out_specs=pl.BlockSpec((ROWB, 64), lambda i: (i, 0)),
        out_shape=jax.ShapeDtypeStruct((n, 64), jnp.float32),
    )(x, u)


def _zea_body(x_ref, w_ref, wa_ref, z_ref, ea_ref, mb_ref, *, elu_in, pairs_in, emit_z):
    i = pl.program_id(0)
    hqs = []
    for q in range(4):
        hq = x_ref[q] if pairs_in else x_ref[:, 16 * q:16 * (q + 1)]
        if elu_in:
            hq = _elu(hq)
        hqs.append(hq)
    wa = wa_ref[...]
    ea = sum(jnp.dot(hqs[q], wa[16 * q:16 * (q + 1), :], preferred_element_type=jnp.float32) for q in range(4))
    ea_ref[...] = ea

    @pl.when(i == 0)
    def _():
        mb_ref[...] = jnp.full(mb_ref.shape, -3e38, jnp.float32)

    mb_ref[...] = jnp.maximum(mb_ref[...], jnp.max(ea, axis=0, keepdims=True))
    if emit_z:
        w = w_ref[...]
        for p in range(4):
            z_ref[p] = sum(jnp.dot(hqs[q], w[16 * q:16 * (q + 1), 16 * p:16 * (p + 1)], preferred_element_type=jnp.float32) for q in range(4))


@functools.lru_cache(maxsize=None)
def _zea_call(n, m, elu_in, pairs_in, emit_z):
    body = functools.partial(_zea_body, elu_in=elu_in, pairs_in=pairs_in, emit_z=emit_z)
    if pairs_in:
        xspec = pl.BlockSpec((4, ROWB, 16), lambda i: (0, i, 0))
    else:
        xspec = pl.BlockSpec((ROWB, 64), lambda i: (i, 0))
    return pl.pallas_call(
        body,
        grid=(n // ROWB,),
        in_specs=[xspec,
                  pl.BlockSpec((64, 64), lambda i: (0, 0)),
                  pl.BlockSpec((64, m), lambda i: (0, 0))],
        out_specs=[pl.BlockSpec((4, ROWB, 16), lambda i: (0, i, 0)),
                   pl.BlockSpec((ROWB, m), lambda i: (i, 0)),
                   pl.BlockSpec((1, m), lambda i: (0, 0))],
        out_shape=[jax.ShapeDtypeStruct((4, n, 16), jnp.float32),
                   jax.ShapeDtypeStruct((n, m), jnp.float32),
                   jax.ShapeDtypeStruct((1, m), jnp.float32)],
    )


def _zea(x, w, wa, elu_in, pairs_in, emit_z, n):
    z4, ea, mb = _zea_call(n, wa.shape[1], elu_in, pairs_in, emit_z)(x, w, wa)
    return (z4 if emit_z else None), ea, mb


def _elu_pairs_body(x_ref, o_ref):
    o_ref[...] = _elu(x_ref[...])


def _elu_pairs(acc4, n):
    return pl.pallas_call(
        _elu_pairs_body,
        grid=(n // ROWB, 4),
        in_specs=[pl.BlockSpec((1, ROWB, 16), lambda i, pb: (pb, i, 0))],
        out_specs=pl.BlockSpec((1, ROWB, 16), lambda i, pb: (pb, i, 0)),
        out_shape=jax.ShapeDtypeStruct((4, n, 16), jnp.float32),
    )(acc4)


def _expand_body(a_ref, o_ref):
    a = a_ref[...]
    rb = a.shape[0]
    for q in range(4):
        c0 = jnp.broadcast_to(a[:, 2 * q:2 * q + 1], (rb, 8))
        c1 = jnp.broadcast_to(a[:, 2 * q + 1:2 * q + 2], (rb, 8))
        o_ref[q] = jnp.concatenate([c0, c1], axis=1)


def _expand(al, epad):
    rb = 2048
    return pl.pallas_call(
        _expand_body,
        grid=(epad // rb,),
        in_specs=[pl.BlockSpec((rb, 16), lambda i: (i, 0))],
        out_specs=pl.BlockSpec((4, rb, 16), lambda i: (0, i, 0)),
        out_shape=jax.ShapeDtypeStruct((4, epad, 16), jnp.float32),
    )(al)


def _fc_body(bd_ref, ms_ref, cnt_ref, mx_ref, w0, b0, w1, b1, w2, b2, o_ref):
    hb = jnp.concatenate([_elu(bd_ref[q]) for q in range(4)], axis=1)
    cnt = jnp.maximum(cnt_ref[:, 0:1], 1.0)
    mean = jnp.concatenate([ms_ref[q] for q in range(4)], axis=1) / cnt
    mxv = jnp.concatenate([mx_ref[q] for q in range(4)], axis=1)
    v = jnp.concatenate([hb, mean, mxv], axis=1)
    h = jnp.maximum(jnp.dot(v, w0[...], preferred_element_type=jnp.float32) + b0[...], 0.0)
    h = jnp.maximum(jnp.dot(h, w1[...], preferred_element_type=jnp.float32) + b1[...], 0.0)
    o_ref[...] = jnp.dot(h, w2[...], preferred_element_type=jnp.float32) + b2[...]


def _fc(bond4, ms4, cnt16, mx4, p):
    n = NBN
    return pl.pallas_call(
        _fc_body,
        grid=(n // ROWB,),
        in_specs=[pl.BlockSpec((4, ROWB, 16), lambda i: (0, i, 0)),
                  pl.BlockSpec((4, ROWB, 16), lambda i: (0, i, 0)),
                  pl.BlockSpec((ROWB, 16), lambda i: (i, 0)),
                  pl.BlockSpec((4, ROWB, 16), lambda i: (0, i, 0)),
                  pl.BlockSpec((192, 64), lambda i: (0, 0)),
                  pl.BlockSpec((64,), lambda i: (0,)),
                  pl.BlockSpec((64, 64), lambda i: (0, 0)),
                  pl.BlockSpec((64,), lambda i: (0,)),
                  pl.BlockSpec((64, 1), lambda i: (0, 0)),
                  pl.BlockSpec((1,), lambda i: (0,))],
        out_specs=pl.BlockSpec((ROWB, 1), lambda i: (i, 0)),
        out_shape=jax.ShapeDtypeStruct((n, 1), jnp.float32),
    )(bond4, ms4, cnt16, mx4, p["fc0_W"], p["fc0_b"], p["fc1_W"], p["fc1_b"], p["fc2_W"], p["fc2_b"])


_MESH = None


def _mesh():
    global _MESH
    if _MESH is None:
        _MESH = plsc.VectorSubcoreMesh(core_axis_name="c", subcore_axis_name="s")
    return _MESH


def _fill_rows(ref, nrows, val):
    def body(i, _):
        ref[i, :] = jnp.full((16,), val, jnp.float32)
        return 0
    lax.fori_loop(0, nrows, body, 0)


def _zero_shared(sh_ref, zbuf, s, ndp):
    rpt = ndp // 16
    for off in range(0, rpt, 512):
        sz = min(512, rpt - off)
        pltpu.sync_copy(zbuf.at[pl.ds(0, sz)], sh_ref.at[pl.ds(s * rpt + off, sz)])


def _copy_shared_out(sh_ref, out_hbm, s, ndp, row0):
    rpt = ndp // 16
    for off in range(0, rpt, 512):
        sz = min(512, rpt - off)
        pltpu.sync_copy(sh_ref.at[pl.ds(s * rpt + off, sz)], out_hbm.at[pl.ds(row0 + s * rpt + off, sz)])


def _edge_groups(ept, grp_fn, gsz=512):
    nfull = ept // gsz
    rem = (ept - nfull * gsz) // 128
    if nfull:
        def body(g, _):
            grp_fn(g * gsz, gsz // 128)
            return 0
        lax.fori_loop(0, nfull, body, 0)
    if rem:
        grp_fn(nfull * gsz, rem)


def _passA_core(s, src_h, dst_h, es_h, ed_h, mb_h, ex_h, den_h, den_sh,
                srcv, dstv, esg, edg, exv, mbv, sem, ept, ndp):
    _fill_rows(exv, 512, 0.0)
    _zero_shared(den_sh, exv, s, ndp)
    pltpu.sync_copy(mb_h, mbv)
    plsc.subcore_barrier()
    mb = mbv[...]
    ebase = s * ept

    def grp(goff, js):
        st = ebase + goff
        hs = []
        for j in range(js):
            hs.append(pltpu.async_copy(src_h.at[pl.ds(st + j * 128, 128)], srcv.at[j], sem))
            hs.append(pltpu.async_copy(dst_h.at[pl.ds(st + j * 128, 128)], dstv.at[j], sem))
        for h in hs:
            h.wait()
        hs = []
        for j in range(js):
            hs.append(pltpu.async_copy(es_h.at[srcv.at[j]], esg.at[pl.ds(j * 128, 128)], sem))
            hs.append(pltpu.async_copy(ed_h.at[dstv.at[j]], edg.at[pl.ds(j * 128, 128)], sem))
        for h in hs:
            h.wait()

        def cb(i, _):
            ev = esg[i, :] + edg[i, :]
            ev = jnp.maximum(ev, 0.2 * ev) - mb
            exv[i, :] = jnp.exp(ev)
            return 0
        lax.fori_loop(0, js * 128, cb, 0)
        hs = []
        for j in range(js):
            hs.append(pltpu.async_copy(exv.at[pl.ds(j * 128, 128)], den_sh.at[dstv.at[j]], sem, add=True))
        for h in hs:
            h.wait()
        pltpu.async_copy(exv.at[pl.ds(0, js * 128)], ex_h.at[pl.ds(st, js * 128)], sem).wait()

    _edge_groups(ept, grp)
    plsc.subcore_barrier()
    _copy_shared_out(den_sh, den_h, s, ndp, 0)


@functools.lru_cache(maxsize=None)
def _passA_call(e0, e1, ndp):
    def body(src0, dst0, es0, ed0, mb0, src1, dst1, es1, ed1, mb1,
             ex0, den0, ex1, den1, den_sh, srcv, dstv, esg, edg, exv, mbv, sem):
        c = lax.axis_index("c")
        s = lax.axis_index("s")

        @pl.when(c == 0)
        def _():
            _passA_core(s, src0, dst0, es0, ed0, mb0, ex0, den0, den_sh,
                        srcv, dstv, esg, edg, exv, mbv, sem, e0 // 16, ndp)

        @pl.when(c == 1)
        def _():
            _passA_core(s, src1, dst1, es1, ed1, mb1, ex1, den1, den_sh,
                        srcv, dstv, esg, edg, exv, mbv, sem, e1 // 16, ndp)

    return pl.kernel(
        body,
        out_type=[jax.ShapeDtypeStruct((e0, 16), jnp.float32),
                  jax.ShapeDtypeStruct((ndp, 16), jnp.float32),
                  jax.ShapeDtypeStruct((e1, 16), jnp.float32),
                  jax.ShapeDtypeStruct((ndp, 16), jnp.float32)],
        mesh=_mesh(),
        compiler_params=pltpu.CompilerParams(use_tc_tiling_on_sc=False),
        scratch_types=[pltpu.VMEM_SHARED((ndp, 16), jnp.float32),
                       pltpu.VMEM((4, 128), jnp.int32),
                       pltpu.VMEM((4, 128), jnp.int32),
                       pltpu.VMEM((512, 16), jnp.float32),
                       pltpu.VMEM((512, 16), jnp.float32),
                       pltpu.VMEM((512, 16), jnp.float32),
                       pltpu.VMEM((16,), jnp.float32),
                       pltpu.SemaphoreType.DMA],
        name="hgat_passA",
    )


def _passA2_core(s, dst_h, ex_h, den_h, al_h, dstf, dstv, exv, deng, zbuf_unused, sem, ept):
    ebase = s * ept

    def grp(goff, js):
        st = ebase + goff
        hs = [pltpu.async_copy(dst_h.at[pl.ds(st, js * 128)], dstf.at[pl.ds(0, js * 128)], sem),
              pltpu.async_copy(ex_h.at[pl.ds(st, js * 128)], exv.at[pl.ds(0, js * 128)], sem)]
        for h in hs:
            h.wait()
        hs = []
        for j in range(js):
            hs.append(pltpu.async_copy(den_h.at[dstf.at[pl.ds(j * 128, 128)]], deng.at[pl.ds(j * 128, 128)], sem))
        for h in hs:
            h.wait()

        def cb(i, _):
            exv[i, :] = exv[i, :] / (deng[i, :] + 1e-9)
            return 0
        lax.fori_loop(0, js * 128, cb, 0)
        pltpu.async_copy(exv.at[pl.ds(0, js * 128)], al_h.at[pl.ds(st, js * 128)], sem).wait()

    _edge_groups(ept, grp, gsz=1024)


@functools.lru_cache(maxsize=None)
def _passA2_call(e0, e1, ndp):
    def body(dst0, ex0, den0, dst1, ex1, den1, al0, al1, dstf, dstv, exv, deng, sem):
        c = lax.axis_index("c")
        s = lax.axis_index("s")

        @pl.when(c == 0)
        def _():
            _passA2_core(s, dst0, ex0, den0, al0, dstf, dstv, exv, deng, None, sem, e0 // 16)

        @pl.when(c == 1)
        def _():
            _passA2_core(s, dst1, ex1, den1, al1, dstf, dstv, exv, deng, None, sem, e1 // 16)

    return pl.kernel(
        body,
        out_type=[jax.ShapeDtypeStruct((e0, 16), jnp.float32),
                  jax.ShapeDtypeStruct((e1, 16), jnp.float32)],
        mesh=_mesh(),
        compiler_params=pltpu.CompilerParams(use_tc_tiling_on_sc=False),
        scratch_types=[pltpu.VMEM((1024,), jnp.int32),
                       pltpu.VMEM((8, 128), jnp.int32),
                       pltpu.VMEM((1024, 16), jnp.float32),
                       pltpu.VMEM((1024, 16), jnp.float32),
                       pltpu.SemaphoreType.DMA],
        name="hgat_passA2",
    )


def _passB_edges(s, p, src_h, dst_h, al_h, z4_h, acc_sh, srcf, dstv, av, zg, cv, sem, ept, ns, epad, has_alpha):
    ebase = s * ept
    pns = p * ns
    pep = p * epad

    def grp(goff, js):
        st = ebase + goff
        hs = [pltpu.async_copy(src_h.at[pl.ds(st, js * 128)], srcf.at[pl.ds(0, js * 128)], sem)]
        for j in range(js):
            hs.append(pltpu.async_copy(dst_h.at[pl.ds(st + j * 128, 128)], dstv.at[j], sem))
        if has_alpha:
            hs.append(pltpu.async_copy(al_h.at[pl.ds(pep + st, js * 128)], av.at[pl.ds(0, js * 128)], sem))
        for h in hs:
            h.wait()

        def adj(v, _):
            srcf[pl.ds(v * 16, 16)] = srcf[pl.ds(v * 16, 16)] + pns
            return 0
        lax.fori_loop(0, js * 8, adj, 0)
        hs = []
        for j in range(js):
            hs.append(pltpu.async_copy(z4_h.at[srcf.at[pl.ds(j * 128, 128)]], zg.at[pl.ds(j * 128, 128)], sem))
        for h in hs:
            h.wait()
        if has_alpha:
            def cb(i, _):
                cv[i, :] = av[i, :] * zg[i, :]
                return 0
            lax.fori_loop(0, js * 128, cb, 0)
            sbuf = cv
        else:
            sbuf = zg
        hs = []
        for j in range(js):
            hs.append(pltpu.async_copy(sbuf.at[pl.ds(j * 128, 128)], acc_sh.at[dstv.at[j]], sem, add=True))
        for h in hs:
            h.wait()

    _edge_groups(ept, grp)


@functools.lru_cache(maxsize=None)
def _passB_call(espec, ndp, has_alpha):
    def body(*refs):
        ins = []
        k = 0
        for _ in espec:
            ins.append(refs[k:k + 4])
            k += 4
        out4 = refs[k]
        acc_sh, srcf, dstv, av, zg, cv, sem = refs[k + 1:]
        c = lax.axis_index("c")
        s = lax.axis_index("s")
        for pp in range(2):
            p = 2 * c + pp
            _fill_rows(cv, 512, 0.0)
            _zero_shared(acc_sh, cv, s, ndp)
            plsc.subcore_barrier()
            for (src_h, dst_h, al_h, z4_h), (epad, ns) in zip(ins, espec):
                _passB_edges(s, p, src_h, dst_h, al_h, z4_h, acc_sh, srcf, dstv, av, zg, cv,
                             sem, epad // 16, ns, epad, has_alpha)
            plsc.subcore_barrier()
            _copy_shared_out(acc_sh, out4, s, ndp, p * ndp)
            plsc.subcore_barrier()

    return pl.kernel(
        body,
        out_type=[jax.ShapeDtypeStruct((4 * ndp, 16), jnp.float32)],
        mesh=_mesh(),
        compiler_params=pltpu.CompilerParams(use_tc_tiling_on_sc=False),
        scratch_types=[pltpu.VMEM_SHARED((ndp, 16), jnp.float32),
                       pltpu.VMEM((512,), jnp.int32),
                       pltpu.VMEM((4, 128), jnp.int32),
                       pltpu.VMEM((512, 16), jnp.float32),
                       pltpu.VMEM((512, 16), jnp.float32),
                       pltpu.VMEM((512, 16), jnp.float32),
                       pltpu.SemaphoreType.DMA],
        name="hgat_passB",
    )


def _passB(etys, ndp, has_alpha):
    espec = tuple((e[0].shape[0], e[4]) for e in etys)
    args = []
    for (srcp, dstp, al, z4f, _ns) in etys:
        args += [srcp, dstp, al, z4f]
    return _passB_call(espec, ndp, has_alpha)(*args)[0]


@functools.lru_cache(maxsize=None)
def _cnt_call(epad, ndp):
    def body(dst_h, cnt_h, cnt_sh, dstv, onesb, zbuf, sem):
        c = lax.axis_index("c")
        s = lax.axis_index("s")

        @pl.when(c == 0)
        def _():
            _fill_rows(zbuf, 512, 0.0)
            _zero_shared(cnt_sh, zbuf, s, ndp)
            _fill_rows(onesb, 128, 1.0)
            plsc.subcore_barrier()
            ept = epad // 16
            ebase = s * ept

            def grp(goff, js):
                st = ebase + goff
                hs = []
                for j in range(js):
                    hs.append(pltpu.async_copy(dst_h.at[pl.ds(st + j * 128, 128)], dstv.at[j], sem))
                for h in hs:
                    h.wait()
                hs = []
                for j in range(js):
                    hs.append(pltpu.async_copy(onesb.at[pl.ds(0, 128)], cnt_sh.at[dstv.at[j]], sem, add=True))
                for h in hs:
                    h.wait()

            _edge_groups(ept, grp, gsz=1024)
            plsc.subcore_barrier()
            _copy_shared_out(cnt_sh, cnt_h, s, ndp, 0)

        @pl.when(c == 1)
        def _():
            plsc.subcore_barrier()
            plsc.subcore_barrier()

    return pl.kernel(
        body,
        out_type=[jax.ShapeDtypeStruct((ndp, 16), jnp.float32)],
        mesh=_mesh(),
        compiler_params=pltpu.CompilerParams(use_tc_tiling_on_sc=False),
        scratch_types=[pltpu.VMEM_SHARED((ndp, 16), jnp.float32),
                       pltpu.VMEM((8, 128), jnp.int32),
                       pltpu.VMEM((128, 16), jnp.float32),
                       pltpu.VMEM((512, 16), jnp.float32),
                       pltpu.SemaphoreType.DMA],
        name="hgat_cnt",
    )


_RNG = 3128
_NRR = 32 * _RNG


@functools.lru_cache(maxsize=None)
def _max_call(epad):
    def body(src_h, dst_h, hat_h, mx_h, sv, dv, idxv, zg, mb0, mb1, sem):
        c = lax.axis_index("c")
        s = lax.axis_index("s")
        w = c * 16 + s
        base = w * _RNG

        for p0 in (0, 2):
            _fill_rows(mb0, _RNG + 8, -1e30)
            _fill_rows(mb1, _RNG + 8, -1e30)

            def chunk(g, _):
                h1 = pltpu.async_copy(src_h.at[pl.ds(g * 2048, 2048)], sv, sem)
                h2 = pltpu.async_copy(dst_h.at[pl.ds(g * 2048, 2048)], dv, sem)
                h1.wait()
                h2.wait()

                def grp(t, _2):
                    for r in range(8):
                        s16 = sv[pl.ds(t * 128 + r * 16, 16)]
                        d16 = dv[pl.ds(t * 128 + r * 16, 16)]
                        m = (d16 >= base) & (d16 < base + _RNG)
                        gi = jnp.where(m, s16, 0)
                        idxv[pl.ds(r * 16, 16)] = gi + (p0 * NAT)
                        idxv[pl.ds(128 + r * 16, 16)] = gi + ((p0 + 1) * NAT)
                    pltpu.async_copy(hat_h.at[idxv], zg, sem).wait()
                    for r in range(8):
                        d16 = dv[pl.ds(t * 128 + r * 16, 16)]
                        m = (d16 >= base) & (d16 < base + _RNG)
                        off16 = jnp.where(m, d16 - base, _RNG)
                        for q in range(16):
                            off = off16[q]
                            mb0[off, :] = jnp.maximum(mb0[off, :], zg[r * 16 + q, :])
                            mb1[off, :] = jnp.maximum(mb1[off, :], zg[128 + r * 16 + q, :])
                    return 0
                lax.fori_loop(0, 16, grp, 0)
                return 0
            lax.fori_loop(0, epad // 2048, chunk, 0)

            for mb, prow in ((mb0, p0), (mb1, p0 + 1)):
                def fz(i, _, mb=mb):
                    vv = mb[i, :]
                    mb[i, :] = jnp.where(vv < -1e29, 0.0, vv)
                    return 0
                lax.fori_loop(0, _RNG, fz, 0)
                for off in range(0, _RNG, 512):
                    sz = min(512, _RNG - off)
                    pltpu.sync_copy(mb.at[pl.ds(off, sz)], mx_h.at[pl.ds(prow * _NRR + base + off, sz)])

    return pl.kernel(
        body,
        out_type=[jax.ShapeDtypeStruct((4 * _NRR, 16), jnp.float32)],
        mesh=_mesh(),
        compiler_params=pltpu.CompilerParams(use_tc_tiling_on_sc=False),
        scratch_types=[pltpu.VMEM((2048,), jnp.int32),
                       pltpu.VMEM((2048,), jnp.int32),
                       pltpu.VMEM((256,), jnp.int32),
                       pltpu.VMEM((256, 16), jnp.float32),
                       pltpu.VMEM((_RNG + 8, 16), jnp.float32),
                       pltpu.VMEM((_RNG + 8, 16), jnp.float32),
                       pltpu.SemaphoreType.DMA],
        name="hgat_max",
    )


def _adiag(a):
    m = (a[:, :, None] * jnp.eye(8, dtype=a.dtype)[:, None, :]).reshape(64, 8)
    return jnp.concatenate([m, jnp.zeros((64, 8), a.dtype)], axis=1)


def _mbvec(mbs, mbd):
    return jnp.concatenate([mbs[0, 0:8] + mbd[0, 0:8], jnp.zeros((8,), jnp.float32)])


def _pad_edges(src, dst, nd):
    e = src.shape[0]
    ep = _rup(e, 2048)
    s = jnp.concatenate([src.astype(jnp.int32), jnp.zeros((ep - e,), jnp.int32)])
    d = jnp.concatenate([dst.astype(jnp.int32), jnp.full((ep - e,), nd, jnp.int32)])
    return s, d


def kernel(x_atom, x_bond, x_glob, a2b_src, a2b_dst, b2a_src, b2a_dst, g2a_src, g2a_dst, a2g_src, a2g_dst, g2b_src, g2b_dst, b2g_src, b2g_dst, params):
    p = params
    ed_raw = {"a2b": (a2b_src, a2b_dst, NAT, NBN, NBNP), "b2a": (b2a_src, b2a_dst, NBN, NAT, NATP),
              "g2a": (g2a_src, g2a_dst, NGL, NAT, NATP), "a2g": (a2g_src, a2g_dst, NAT, NGL, NGLP),
              "g2b": (g2b_src, g2b_dst, NGL, NBN, NBNP), "b2g": (b2g_src, b2g_dst, NBN, NGL, NGLP)}
    E = {}
    for et, (src, dst, ns, nd, ndp) in ed_raw.items():
        sp, dp = _pad_edges(src, dst, nd)
        E[et] = (sp, dp, ns, nd, ndp)

    hA = _mm_init(x_atom, p["U_atom"])
    hB = _mm_init(x_bond, p["U_bond"])
    hG = _mm_init(x_glob, p["U_glob"])
    hA = ("flat", hA)
    hB = ("flat", hB)
    hG = ("flat", hG)

    def run_zea(h, w, mats, emit_z, n):
        kind, x = h
        wa = w @ jnp.concatenate(mats, axis=1)
        return _zea(x, w, wa, elu_in=(kind == "pairs"), pairs_in=(kind == "pairs"), emit_z=emit_z, n=n)

    def attn(update_nd, ndp, etys):
        a_args = []
        for (et, es, ed, mb, _z4) in etys:
            sp, dp, _, _, _ = E[et]
            a_args += [sp, dp, es, ed, mb]
        e0 = E[etys[0][0]][0].shape[0]
        e1 = E[etys[1][0]][0].shape[0]
        ex0, den0, ex1, den1 = _passA_call(e0, e1, ndp)(*a_args)
        al0, al1 = _passA2_call(e0, e1, ndp)(
            E[etys[0][0]][1], ex0, den0, E[etys[1][0]][1], ex1, den1)
        al0 = _expand(al0, e0).reshape(4 * e0, 16)
        al1 = _expand(al1, e1).reshape(4 * e1, 16)
        bargs = []
        for (et, al, z4) in ((etys[0][0], al0, etys[0][4]), (etys[1][0], al1, etys[1][4])):
            sp, dp, ns, _, _ = E[et]
            bargs.append((sp, dp, al, z4.reshape(4 * ns, 16), ns))
        out = _passB(bargs, ndp, True)
        return out.reshape(4, ndp, 16)

    for l in range(3):
        W = {nt: p["W%d_%s" % (l, nt)] for nt in ("atom", "bond", "global")}
        As = {et: _adiag(p["asrc%d_%s" % (l, et)]) for et in ("b2a", "g2a", "a2b", "g2b", "a2g", "b2g")}
        Ad = {et: _adiag(p["adst%d_%s" % (l, et)]) for et in ("b2a", "g2a", "a2b", "g2b", "a2g", "b2g")}

        _, ea_a0, mb_a0 = run_zea(hA, W["atom"], [Ad["b2a"], Ad["g2a"]], False, NAT)
        z4_b0, ea_b0, mb_b0 = run_zea(hB, W["bond"], [As["b2a"], Ad["a2b"], Ad["g2b"]], True, NBN)
        z4_g0, ea_g0, mb_g0 = run_zea(hG, W["global"], [As["g2a"], As["g2b"], Ad["a2g"], Ad["b2g"]], True, NGL)

        acc_a = attn(NAT, NATP, [
            ("b2a", ea_b0[:, 0:16], ea_a0[:, 0:16], _mbvec(mb_b0[:, 0:16], mb_a0[:, 0:16]), z4_b0),
            ("g2a", ea_g0[:, 0:16], ea_a0[:, 16:32], _mbvec(mb_g0[:, 0:16], mb_a0[:, 16:32]), z4_g0)])
        hA = ("pairs", acc_a)

        z4_a1, ea_a1, mb_a1 = run_zea(hA, W["atom"], [As["a2b"], As["a2g"]], True, NAT)
        acc_b = attn(NBN, NBNP, [
            ("a2b", ea_a1[:, 0:16], ea_b0[:, 16:32], _mbvec(mb_a1[:, 0:16], mb_b0[:, 16:32]), z4_a1),
            ("g2b", ea_g0[:, 16:32], ea_b0[:, 32:48], _mbvec(mb_g0[:, 16:32], mb_b0[:, 32:48]), z4_g0)])
        hB = ("pairs", acc_b)

        z4_b1, ea_b1, mb_b1 = run_zea(hB, W["bond"], [As["b2g"]], True, NBN)
        acc_g = attn(NGL, NGLP, [
            ("a2g", ea_a1[:, 16:32], ea_g0[:, 32:48], _mbvec(mb_a1[:, 16:32], mb_g0[:, 32:48]), z4_a1),
            ("b2g", ea_b1[:, 0:16], ea_g0[:, 48:64], _mbvec(mb_b1[:, 0:16], mb_g0[:, 48:64]), z4_b1)])
        hG = ("pairs", acc_g)

    hat4 = _elu_pairs(hA[1], NAT)
    hat4f = hat4.reshape(4 * NAT, 16)
    sp_ab, dp_ab = E["a2b"][0], E["a2b"][1]
    epad_ab = sp_ab.shape[0]
    cnt16 = _cnt_call(epad_ab, NBNP)(dp_ab)[0]
    ms4 = _passB([(sp_ab, dp_ab, dp_ab, hat4f, NAT)], NBNP, False).reshape(4, NBNP, 16)
    mx4 = _max_call(epad_ab)(sp_ab, dp_ab, hat4f)[0].reshape(4, _NRR, 16)
    out = _fc(hB[1], ms4, cnt16, mx4, p)
    return out.reshape(-1)


# speedup vs baseline: 4.5867x; 1.0008x over previous
import functools

import jax
import jax.numpy as jnp
from jax import lax
from jax.experimental import pallas as pl
from jax.experimental.pallas import tpu as pltpu
from jax.experimental.pallas import tpu_sc as plsc

NAT, NBN, NGL = 100000, 100000, 5000
NH, NDH, HID = 8, 8, 64
ROWB = 1000


def _rup(x, m):
    return ((x + m - 1) // m) * m


def _np(n):
    return _rup(n + 8, 128)


NATP, NBNP, NGLP = _np(NAT), _np(NBN), _np(NGL)


def _elu(x):
    return jnp.where(x > 0, x, jnp.exp(x) - 1.0)


def _mm_init_body(x_ref, u_ref, o_ref):
    o_ref[...] = jnp.dot(x_ref[...], u_ref[...], preferred_element_type=jnp.float32)


def _mm_init(x, u):
    n = x.shape[0]
    return pl.pallas_call(
        _mm_init_body,
        grid=(n // ROWB,),
        in_specs=[pl.BlockSpec((ROWB, x.shape[1]), lambda i: (i, 0)),
                  pl.BlockSpec(u.shape, lambda i: (0, 0))],
        out_specs=pl.BlockSpec((ROWB, 64), lambda i: (i, 0)),
        out_shape=jax.ShapeDtypeStruct((n, 64), jnp.float32),
    )(x, u)


def _zea_body(x_ref, w_ref, wa_ref, z_ref, ea_ref, mb_ref, *, elu_in, pairs_in, emit_z):
    i = pl.program_id(0)
    hqs = []
    for q in range(4):
        hq = x_ref[q] if pairs_in else x_ref[:, 16 * q:16 * (q + 1)]
        if elu_in:
            hq = _elu(hq)
        hqs.append(hq)
    wa = wa_ref[...]
    ea = sum(jnp.dot(hqs[q], wa[16 * q:16 * (q + 1), :], preferred_element_type=jnp.float32) for q in range(4))
    ea_ref[...] = ea

    @pl.when(i == 0)
    def _():
        mb_ref[...] = jnp.full(mb_ref.shape, -3e38, jnp.float32)

    mb_ref[...] = jnp.maximum(mb_ref[...], jnp.max(ea, axis=0, keepdims=True))
    if emit_z:
        w = w_ref[...]
        for p in range(4):
            z_ref[p] = sum(jnp.dot(hqs[q], w[16 * q:16 * (q + 1), 16 * p:16 * (p + 1)], preferred_element_type=jnp.float32) for q in range(4))


@functools.lru_cache(maxsize=None)
def _zea_call(n, m, elu_in, pairs_in, emit_z):
    body = functools.partial(_zea_body, elu_in=elu_in, pairs_in=pairs_in, emit_z=emit_z)
    if pairs_in:
        xspec = pl.BlockSpec((4, ROWB, 16), lambda i: (0, i, 0))
    else:
        xspec = pl.BlockSpec((ROWB, 64), lambda i: (i, 0))
    return pl.pallas_call(
        body,
        grid=(n // ROWB,),
        in_specs=[xspec,
                  pl.BlockSpec((64, 64), lambda i: (0, 0)),
                  pl.BlockSpec((64, m), lambda i: (0, 0))],
        out_specs=[pl.BlockSpec((4, ROWB, 16), lambda i: (0, i, 0)),
                   pl.BlockSpec((ROWB, m), lambda i: (i, 0)),
                   pl.BlockSpec((1, m), lambda i: (0, 0))],
        out_shape=[jax.ShapeDtypeStruct((4, n, 16), jnp.float32),
                   jax.ShapeDtypeStruct((n, m), jnp.float32),
                   jax.ShapeDtypeStruct((1, m), jnp.float32)],
    )


def _zea(x, w, wa, elu_in, pairs_in, emit_z, n):
    z4, ea, mb = _zea_call(n, wa.shape[1], elu_in, pairs_in, emit_z)(x, w, wa)
    return (z4 if emit_z else None), ea, mb


def _elu_pairs_body(x_ref, o_ref):
    o_ref[...] = _elu(x_ref[...])


def _elu_pairs(acc4, n):
    return pl.pallas_call(
        _elu_pairs_body,
        grid=(n // ROWB, 4),
        in_specs=[pl.BlockSpec((1, ROWB, 16), lambda i, pb: (pb, i, 0))],
        out_specs=pl.BlockSpec((1, ROWB, 16), lambda i, pb: (pb, i, 0)),
        out_shape=jax.ShapeDtypeStruct((4, n, 16), jnp.float32),
    )(acc4)


def _expand_body(a_ref, o_ref):
    a = a_ref[...]
    rb = a.shape[0]
    for q in range(4):
        c0 = jnp.broadcast_to(a[:, 2 * q:2 * q + 1], (rb, 8))
        c1 = jnp.broadcast_to(a[:, 2 * q + 1:2 * q + 2], (rb, 8))
        o_ref[q] = jnp.concatenate([c0, c1], axis=1)


def _expand(al, epad):
    rb = 2048
    return pl.pallas_call(
        _expand_body,
        grid=(epad // rb,),
        in_specs=[pl.BlockSpec((rb, 16), lambda i: (i, 0))],
        out_specs=pl.BlockSpec((4, rb, 16), lambda i: (0, i, 0)),
        out_shape=jax.ShapeDtypeStruct((4, epad, 16), jnp.float32),
    )(al)


def _fc_body(bd_ref, ms_ref, cnt_ref, mx_ref, w0, b0, w1, b1, w2, b2, o_ref):
    hb = jnp.concatenate([_elu(bd_ref[q]) for q in range(4)], axis=1)
    cnt = jnp.maximum(cnt_ref[:, 0:1], 1.0)
    mean = jnp.concatenate([ms_ref[q] for q in range(4)], axis=1) / cnt
    mxv = jnp.concatenate([mx_ref[q] for q in range(4)], axis=1)
    v = jnp.concatenate([hb, mean, mxv], axis=1)
    h = jnp.maximum(jnp.dot(v, w0[...], preferred_element_type=jnp.float32) + b0[...], 0.0)
    h = jnp.maximum(jnp.dot(h, w1[...], preferred_element_type=jnp.float32) + b1[...], 0.0)
    o_ref[...] = jnp.dot(h, w2[...], preferred_element_type=jnp.float32) + b2[...]


def _fc(bond4, ms4, cnt16, mx4, p):
    n = NBN
    return pl.pallas_call(
        _fc_body,
        grid=(n // ROWB,),
        in_specs=[pl.BlockSpec((4, ROWB, 16), lambda i: (0, i, 0)),
                  pl.BlockSpec((4, ROWB, 16), lambda i: (0, i, 0)),
                  pl.BlockSpec((ROWB, 16), lambda i: (i, 0)),
                  pl.BlockSpec((4, ROWB, 16), lambda i: (0, i, 0)),
                  pl.BlockSpec((192, 64), lambda i: (0, 0)),
                  pl.BlockSpec((64,), lambda i: (0,)),
                  pl.BlockSpec((64, 64), lambda i: (0, 0)),
                  pl.BlockSpec((64,), lambda i: (0,)),
                  pl.BlockSpec((64, 1), lambda i: (0, 0)),
                  pl.BlockSpec((1,), lambda i: (0,))],
        out_specs=pl.BlockSpec((ROWB, 1), lambda i: (i, 0)),
        out_shape=jax.ShapeDtypeStruct((n, 1), jnp.float32),
    )(bond4, ms4, cnt16, mx4, p["fc0_W"], p["fc0_b"], p["fc1_W"], p["fc1_b"], p["fc2_W"], p["fc2_b"])


_MESH = None


def _mesh():
    global _MESH
    if _MESH is None:
        _MESH = plsc.VectorSubcoreMesh(core_axis_name="c", subcore_axis_name="s")
    return _MESH


def _fill_rows(ref, nrows, val):
    def body(i, _):
        ref[i, :] = jnp.full((16,), val, jnp.float32)
        return 0
    lax.fori_loop(0, nrows, body, 0)


def _zero_shared(sh_ref, zbuf, s, ndp):
    rpt = ndp // 16
    for off in range(0, rpt, 512):
        sz = min(512, rpt - off)
        pltpu.sync_copy(zbuf.at[pl.ds(0, sz)], sh_ref.at[pl.ds(s * rpt + off, sz)])


def _copy_shared_out(sh_ref, out_hbm, s, ndp, row0):
    rpt = ndp // 16
    for off in range(0, rpt, 512):
        sz = min(512, rpt - off)
        pltpu.sync_copy(sh_ref.at[pl.ds(s * rpt + off, sz)], out_hbm.at[pl.ds(row0 + s * rpt + off, sz)])


def _edge_groups(ept, grp_fn, gsz=512):
    nfull = ept // gsz
    rem = (ept - nfull * gsz) // 128
    if nfull:
        def body(g, _):
            grp_fn(g * gsz, gsz // 128)
            return 0
        lax.fori_loop(0, nfull, body, 0)
    if rem:
        grp_fn(nfull * gsz, rem)


def _passA_core(s, src_h, dst_h, es_h, ed_h, mb_h, ex_h, den_h, den_sh,
                srcv, dstv, esg, edg, exv, mbv, sem, ept, ndp):
    _fill_rows(exv, 512, 0.0)
    _zero_shared(den_sh, exv, s, ndp)
    pltpu.sync_copy(mb_h, mbv)
    plsc.subcore_barrier()
    mb = mbv[...]
    ebase = s * ept

    def grp(goff, js):
        st = ebase + goff
        hs = []
        for j in range(js):
            hs.append(pltpu.async_copy(src_h.at[pl.ds(st + j * 128, 128)], srcv.at[j], sem))
            hs.append(pltpu.async_copy(dst_h.at[pl.ds(st + j * 128, 128)], dstv.at[j], sem))
        for h in hs:
            h.wait()
        hs = []
        for j in range(js):
            hs.append(pltpu.async_copy(es_h.at[srcv.at[j]], esg.at[pl.ds(j * 128, 128)], sem))
            hs.append(pltpu.async_copy(ed_h.at[dstv.at[j]], edg.at[pl.ds(j * 128, 128)], sem))
        for h in hs:
            h.wait()

        def cb(i, _):
            ev = esg[i, :] + edg[i, :]
            ev = jnp.maximum(ev, 0.2 * ev) - mb
            exv[i, :] = jnp.exp(ev)
            return 0
        lax.fori_loop(0, js * 128, cb, 0)
        hs = []
        for j in range(js):
            hs.append(pltpu.async_copy(exv.at[pl.ds(j * 128, 128)], den_sh.at[dstv.at[j]], sem, add=True))
        for h in hs:
            h.wait()
        pltpu.async_copy(exv.at[pl.ds(0, js * 128)], ex_h.at[pl.ds(st, js * 128)], sem).wait()

    _edge_groups(ept, grp)
    plsc.subcore_barrier()
    _copy_shared_out(den_sh, den_h, s, ndp, 0)


@functools.lru_cache(maxsize=None)
def _passA_call(e0, e1, ndp):
    def body(src0, dst0, es0, ed0, mb0, src1, dst1, es1, ed1, mb1,
             ex0, den0, ex1, den1, den_sh, srcv, dstv, esg, edg, exv, mbv, sem):
        c = lax.axis_index("c")
        s = lax.axis_index("s")

        @pl.when(c == 0)
        def _():
            _passA_core(s, src0, dst0, es0, ed0, mb0, ex0, den0, den_sh,
                        srcv, dstv, esg, edg, exv, mbv, sem, e0 // 16, ndp)

        @pl.when(c == 1)
        def _():
            _passA_core(s, src1, dst1, es1, ed1, mb1, ex1, den1, den_sh,
                        srcv, dstv, esg, edg, exv, mbv, sem, e1 // 16, ndp)

    return pl.kernel(
        body,
        out_type=[jax.ShapeDtypeStruct((e0, 16), jnp.float32),
                  jax.ShapeDtypeStruct((ndp, 16), jnp.float32),
                  jax.ShapeDtypeStruct((e1, 16), jnp.float32),
                  jax.ShapeDtypeStruct((ndp, 16), jnp.float32)],
        mesh=_mesh(),
        compiler_params=pltpu.CompilerParams(use_tc_tiling_on_sc=False),
        scratch_types=[pltpu.VMEM_SHARED((ndp, 16), jnp.float32),
                       pltpu.VMEM((4, 128), jnp.int32),
                       pltpu.VMEM((4, 128), jnp.int32),
                       pltpu.VMEM((512, 16), jnp.float32),
                       pltpu.VMEM((512, 16), jnp.float32),
                       pltpu.VMEM((512, 16), jnp.float32),
                       pltpu.VMEM((16,), jnp.float32),
                       pltpu.SemaphoreType.DMA],
        name="hgat_passA",
    )


def _passA2_core(s, dst_h, ex_h, den_h, al_h, dstf, dstv, exv, deng, zbuf_unused, sem, ept):
    ebase = s * ept

    def grp(goff, js):
        st = ebase + goff
        hs = [pltpu.async_copy(dst_h.at[pl.ds(st, js * 128)], dstf.at[pl.ds(0, js * 128)], sem),
              pltpu.async_copy(ex_h.at[pl.ds(st, js * 128)], exv.at[pl.ds(0, js * 128)], sem)]
        for h in hs:
            h.wait()
        hs = []
        for j in range(js):
            hs.append(pltpu.async_copy(den_h.at[dstf.at[pl.ds(j * 128, 128)]], deng.at[pl.ds(j * 128, 128)], sem))
        for h in hs:
            h.wait()

        def cb(i, _):
            exv[i, :] = exv[i, :] / (deng[i, :] + 1e-9)
            return 0
        lax.fori_loop(0, js * 128, cb, 0)
        pltpu.async_copy(exv.at[pl.ds(0, js * 128)], al_h.at[pl.ds(st, js * 128)], sem).wait()

    _edge_groups(ept, grp, gsz=1024)


@functools.lru_cache(maxsize=None)
def _passA2_call(e0, e1, ndp):
    def body(dst0, ex0, den0, dst1, ex1, den1, al0, al1, dstf, dstv, exv, deng, sem):
        c = lax.axis_index("c")
        s = lax.axis_index("s")

        @pl.when(c == 0)
        def _():
            _passA2_core(s, dst0, ex0, den0, al0, dstf, dstv, exv, deng, None, sem, e0 // 16)

        @pl.when(c == 1)
        def _():
            _passA2_core(s, dst1, ex1, den1, al1, dstf, dstv, exv, deng, None, sem, e1 // 16)

    return pl.kernel(
        body,
        out_type=[jax.ShapeDtypeStruct((e0, 16), jnp.float32),
                  jax.ShapeDtypeStruct((e1, 16), jnp.float32)],
        mesh=_mesh(),
        compiler_params=pltpu.CompilerParams(use_tc_tiling_on_sc=False),
        scratch_types=[pltpu.VMEM((1024,), jnp.int32),
                       pltpu.VMEM((8, 128), jnp.int32),
                       pltpu.VMEM((1024, 16), jnp.float32),
                       pltpu.VMEM((1024, 16), jnp.float32),
                       pltpu.SemaphoreType.DMA],
        name="hgat_passA2",
    )


def _passB_edges(s, p, src_h, dst_h, al_h, z4_h, acc_sh, srcf, dstv, av, zg, cv, sem, ept, ns, epad, has_alpha):
    ebase = s * ept
    pns = p * ns
    pep = p * epad

    def grp(goff, js):
        st = ebase + goff
        hs = [pltpu.async_copy(src_h.at[pl.ds(st, js * 128)], srcf.at[pl.ds(0, js * 128)], sem)]
        for j in range(js):
            hs.append(pltpu.async_copy(dst_h.at[pl.ds(st + j * 128, 128)], dstv.at[j], sem))
        if has_alpha:
            hs.append(pltpu.async_copy(al_h.at[pl.ds(pep + st, js * 128)], av.at[pl.ds(0, js * 128)], sem))
        for h in hs:
            h.wait()

        def adj(v, _):
            srcf[pl.ds(v * 16, 16)] = srcf[pl.ds(v * 16, 16)] + pns
            return 0
        lax.fori_loop(0, js * 8, adj, 0)
        hs = []
        for j in range(js):
            hs.append(pltpu.async_copy(z4_h.at[srcf.at[pl.ds(j * 128, 128)]], zg.at[pl.ds(j * 128, 128)], sem))
        for h in hs:
            h.wait()
        if has_alpha:
            def cb(i, _):
                cv[i, :] = av[i, :] * zg[i, :]
                return 0
            lax.fori_loop(0, js * 128, cb, 0)
            sbuf = cv
        else:
            sbuf = zg
        hs = []
        for j in range(js):
            hs.append(pltpu.async_copy(sbuf.at[pl.ds(j * 128, 128)], acc_sh.at[dstv.at[j]], sem, add=True))
        for h in hs:
            h.wait()

    _edge_groups(ept, grp)


@functools.lru_cache(maxsize=None)
def _passB_call(espec, ndp, has_alpha):
    def body(*refs):
        ins = []
        k = 0
        for _ in espec:
            ins.append(refs[k:k + 4])
            k += 4
        out4 = refs[k]
        acc_sh, srcf, dstv, av, zg, cv, sem = refs[k + 1:]
        c = lax.axis_index("c")
        s = lax.axis_index("s")
        for pp in range(2):
            p = 2 * c + pp
            _fill_rows(cv, 512, 0.0)
            _zero_shared(acc_sh, cv, s, ndp)
            plsc.subcore_barrier()
            for (src_h, dst_h, al_h, z4_h), (epad, ns) in zip(ins, espec):
                _passB_edges(s, p, src_h, dst_h, al_h, z4_h, acc_sh, srcf, dstv, av, zg, cv,
                             sem, epad // 16, ns, epad, has_alpha)
            plsc.subcore_barrier()
            _copy_shared_out(acc_sh, out4, s, ndp, p * ndp)
            plsc.subcore_barrier()

    return pl.kernel(
        body,
        out_type=[jax.ShapeDtypeStruct((4 * ndp, 16), jnp.float32)],
        mesh=_mesh(),
        compiler_params=pltpu.CompilerParams(use_tc_tiling_on_sc=False),
        scratch_types=[pltpu.VMEM_SHARED((ndp, 16), jnp.float32),
                       pltpu.VMEM((512,), jnp.int32),
                       pltpu.VMEM((4, 128), jnp.int32),
                       pltpu.VMEM((512, 16), jnp.float32),
                       pltpu.VMEM((512, 16), jnp.float32),
                       pltpu.VMEM((512, 16), jnp.float32),
                       pltpu.SemaphoreType.DMA],
        name="hgat_passB",
    )


def _passB(etys, ndp, has_alpha):
    espec = tuple((e[0].shape[0], e[4]) for e in etys)
    args = []
    for (srcp, dstp, al, z4f, _ns) in etys:
        args += [srcp, dstp, al, z4f]
    return _passB_call(espec, ndp, has_alpha)(*args)[0]


@functools.lru_cache(maxsize=None)
def _cnt_call(epad, ndp):
    def body(dst_h, cnt_h, cnt_sh, dstv, onesb, zbuf, sem):
        c = lax.axis_index("c")
        s = lax.axis_index("s")

        @pl.when(c == 0)
        def _():
            _fill_rows(zbuf, 512, 0.0)
            _zero_shared(cnt_sh, zbuf, s, ndp)
            _fill_rows(onesb, 128, 1.0)
            plsc.subcore_barrier()
            ept = epad // 16
            ebase = s * ept

            def grp(goff, js):
                st = ebase + goff
                hs = []
                for j in range(js):
                    hs.append(pltpu.async_copy(dst_h.at[pl.ds(st + j * 128, 128)], dstv.at[j], sem))
                for h in hs:
                    h.wait()
                hs = []
                for j in range(js):
                    hs.append(pltpu.async_copy(onesb.at[pl.ds(0, 128)], cnt_sh.at[dstv.at[j]], sem, add=True))
                for h in hs:
                    h.wait()

            _edge_groups(ept, grp, gsz=1024)
            plsc.subcore_barrier()
            _copy_shared_out(cnt_sh, cnt_h, s, ndp, 0)

        @pl.when(c == 1)
        def _():
            plsc.subcore_barrier()
            plsc.subcore_barrier()

    return pl.kernel(
        body,
        out_type=[jax.ShapeDtypeStruct((ndp, 16), jnp.float32)],
        mesh=_mesh(),
        compiler_params=pltpu.CompilerParams(use_tc_tiling_on_sc=False),
        scratch_types=[pltpu.VMEM_SHARED((ndp, 16), jnp.float32),
                       pltpu.VMEM((8, 128), jnp.int32),
                       pltpu.VMEM((128, 16), jnp.float32),
                       pltpu.VMEM((512, 16), jnp.float32),
                       pltpu.SemaphoreType.DMA],
        name="hgat_cnt",
    )


_RNG = 3128
_NRR = 32 * _RNG


@functools.lru_cache(maxsize=None)
def _max_call(epad):
    def body(src_h, dst_h, hat_h, mx_h, sv, dv, idxv0, idxv1, zg0, zg1, mb0, mb1, sem):
        c = lax.axis_index("c")
        s = lax.axis_index("s")
        w = c * 16 + s
        base = w * _RNG

        for p0 in (0, 2):
            _fill_rows(mb0, _RNG + 8, -1e30)
            _fill_rows(mb1, _RNG + 8, -1e30)

            def chunk(g, _):
                h1 = pltpu.async_copy(src_h.at[pl.ds(g * 2048, 2048)], sv, sem)
                h2 = pltpu.async_copy(dst_h.at[pl.ds(g * 2048, 2048)], dv, sem)
                h1.wait()
                h2.wait()

                def build(t, ix, zb):
                    for r in range(8):
                        s16 = sv[pl.ds(t * 128 + r * 16, 16)]
                        d16 = dv[pl.ds(t * 128 + r * 16, 16)]
                        m = (d16 >= base) & (d16 < base + _RNG)
                        gi = jnp.where(m, s16, 0)
                        ix[pl.ds(r * 16, 16)] = gi + (p0 * NAT)
                        ix[pl.ds(128 + r * 16, 16)] = gi + ((p0 + 1) * NAT)
                    return pltpu.async_copy(hat_h.at[ix], zb, sem)

                def rmw(t, zb):
                    for r in range(8):
                        d16 = dv[pl.ds(t * 128 + r * 16, 16)]
                        m = (d16 >= base) & (d16 < base + _RNG)
                        off16 = jnp.where(m, d16 - base, _RNG)
                        for q in range(16):
                            off = off16[q]
                            mb0[off, :] = jnp.maximum(mb0[off, :], zb[r * 16 + q, :])
                            mb1[off, :] = jnp.maximum(mb1[off, :], zb[128 + r * 16 + q, :])

                def grp2(u, _2):
                    hA = build(2 * u, idxv0, zg0)
                    hB = build(2 * u + 1, idxv1, zg1)
                    hA.wait()
                    rmw(2 * u, zg0)
                    hB.wait()
                    rmw(2 * u + 1, zg1)
                    return 0
                lax.fori_loop(0, 8, grp2, 0)
                return 0
            lax.fori_loop(0, epad // 2048, chunk, 0)

            for mb, prow in ((mb0, p0), (mb1, p0 + 1)):
                def fz(i, _, mb=mb):
                    vv = mb[i, :]
                    mb[i, :] = jnp.where(vv < -1e29, 0.0, vv)
                    return 0
                lax.fori_loop(0, _RNG, fz, 0)
                for off in range(0, _RNG, 512):
                    sz = min(512, _RNG - off)
                    pltpu.sync_copy(mb.at[pl.ds(off, sz)], mx_h.at[pl.ds(prow * _NRR + base + off, sz)])

    return pl.kernel(
        body,
        out_type=[jax.ShapeDtypeStruct((4 * _NRR, 16), jnp.float32)],
        mesh=_mesh(),
        compiler_params=pltpu.CompilerParams(use_tc_tiling_on_sc=False),
        scratch_types=[pltpu.VMEM((2048,), jnp.int32),
                       pltpu.VMEM((2048,), jnp.int32),
                       pltpu.VMEM((256,), jnp.int32),
                       pltpu.VMEM((256,), jnp.int32),
                       pltpu.VMEM((256, 16), jnp.float32),
                       pltpu.VMEM((256, 16), jnp.float32),
                       pltpu.VMEM((_RNG + 8, 16), jnp.float32),
                       pltpu.VMEM((_RNG + 8, 16), jnp.float32),
                       pltpu.SemaphoreType.DMA],
        name="hgat_max",
    )


def _adiag(a):
    m = (a[:, :, None] * jnp.eye(8, dtype=a.dtype)[:, None, :]).reshape(64, 8)
    return jnp.concatenate([m, jnp.zeros((64, 8), a.dtype)], axis=1)


def _mbvec(mbs, mbd):
    return jnp.concatenate([mbs[0, 0:8] + mbd[0, 0:8], jnp.zeros((8,), jnp.float32)])


def _pad_edges(src, dst, nd):
    e = src.shape[0]
    ep = _rup(e, 2048)
    s = jnp.concatenate([src.astype(jnp.int32), jnp.zeros((ep - e,), jnp.int32)])
    d = jnp.concatenate([dst.astype(jnp.int32), jnp.full((ep - e,), nd, jnp.int32)])
    return s, d


def kernel(x_atom, x_bond, x_glob, a2b_src, a2b_dst, b2a_src, b2a_dst, g2a_src, g2a_dst, a2g_src, a2g_dst, g2b_src, g2b_dst, b2g_src, b2g_dst, params):
    p = params
    ed_raw = {"a2b": (a2b_src, a2b_dst, NAT, NBN, NBNP), "b2a": (b2a_src, b2a_dst, NBN, NAT, NATP),
              "g2a": (g2a_src, g2a_dst, NGL, NAT, NATP), "a2g": (a2g_src, a2g_dst, NAT, NGL, NGLP),
              "g2b": (g2b_src, g2b_dst, NGL, NBN, NBNP), "b2g": (b2g_src, b2g_dst, NBN, NGL, NGLP)}
    E = {}
    for et, (src, dst, ns, nd, ndp) in ed_raw.items():
        sp, dp = _pad_edges(src, dst, nd)
        E[et] = (sp, dp, ns, nd, ndp)

    hA = _mm_init(x_atom, p["U_atom"])
    hB = _mm_init(x_bond, p["U_bond"])
    hG = _mm_init(x_glob, p["U_glob"])
    hA = ("flat", hA)
    hB = ("flat", hB)
    hG = ("flat", hG)

    def run_zea(h, w, mats, emit_z, n):
        kind, x = h
        wa = w @ jnp.concatenate(mats, axis=1)
        return _zea(x, w, wa, elu_in=(kind == "pairs"), pairs_in=(kind == "pairs"), emit_z=emit_z, n=n)

    def attn(update_nd, ndp, etys):
        a_args = []
        for (et, es, ed, mb, _z4) in etys:
            sp, dp, _, _, _ = E[et]
            a_args += [sp, dp, es, ed, mb]
        e0 = E[etys[0][0]][0].shape[0]
        e1 = E[etys[1][0]][0].shape[0]
        ex0, den0, ex1, den1 = _passA_call(e0, e1, ndp)(*a_args)
        al0, al1 = _passA2_call(e0, e1, ndp)(
            E[etys[0][0]][1], ex0, den0, E[etys[1][0]][1], ex1, den1)
        al0 = _expand(al0, e0).reshape(4 * e0, 16)
        al1 = _expand(al1, e1).reshape(4 * e1, 16)
        bargs = []
        for (et, al, z4) in ((etys[0][0], al0, etys[0][4]), (etys[1][0], al1, etys[1][4])):
            sp, dp, ns, _, _ = E[et]
            bargs.append((sp, dp, al, z4.reshape(4 * ns, 16), ns))
        out = _passB(bargs, ndp, True)
        return out.reshape(4, ndp, 16)

    for l in range(3):
        W = {nt: p["W%d_%s" % (l, nt)] for nt in ("atom", "bond", "global")}
        As = {et: _adiag(p["asrc%d_%s" % (l, et)]) for et in ("b2a", "g2a", "a2b", "g2b", "a2g", "b2g")}
        Ad = {et: _adiag(p["adst%d_%s" % (l, et)]) for et in ("b2a", "g2a", "a2b", "g2b", "a2g", "b2g")}

        _, ea_a0, mb_a0 = run_zea(hA, W["atom"], [Ad["b2a"], Ad["g2a"]], False, NAT)
        z4_b0, ea_b0, mb_b0 = run_zea(hB, W["bond"], [As["b2a"], Ad["a2b"], Ad["g2b"]], True, NBN)
        z4_g0, ea_g0, mb_g0 = run_zea(hG, W["global"], [As["g2a"], As["g2b"], Ad["a2g"], Ad["b2g"]], True, NGL)

        acc_a = attn(NAT, NATP, [
            ("b2a", ea_b0[:, 0:16], ea_a0[:, 0:16], _mbvec(mb_b0[:, 0:16], mb_a0[:, 0:16]), z4_b0),
            ("g2a", ea_g0[:, 0:16], ea_a0[:, 16:32], _mbvec(mb_g0[:, 0:16], mb_a0[:, 16:32]), z4_g0)])
        hA = ("pairs", acc_a)

        z4_a1, ea_a1, mb_a1 = run_zea(hA, W["atom"], [As["a2b"], As["a2g"]], True, NAT)
        acc_b = attn(NBN, NBNP, [
            ("a2b", ea_a1[:, 0:16], ea_b0[:, 16:32], _mbvec(mb_a1[:, 0:16], mb_b0[:, 16:32]), z4_a1),
            ("g2b", ea_g0[:, 16:32], ea_b0[:, 32:48], _mbvec(mb_g0[:, 16:32], mb_b0[:, 32:48]), z4_g0)])
        hB = ("pairs", acc_b)

        z4_b1, ea_b1, mb_b1 = run_zea(hB, W["bond"], [As["b2g"]], True, NBN)
        acc_g = attn(NGL, NGLP, [
            ("a2g", ea_a1[:, 16:32], ea_g0[:, 32:48], _mbvec(mb_a1[:, 16:32], mb_g0[:, 32:48]), z4_a1),
            ("b2g", ea_b1[:, 0:16], ea_g0[:, 48:64], _mbvec(mb_b1[:, 0:16], mb_g0[:, 48:64]), z4_b1)])
        hG = ("pairs", acc_g)

    hat4 = _elu_pairs(hA[1], NAT)
    hat4f = hat4.reshape(4 * NAT, 16)
    sp_ab, dp_ab = E["a2b"][0], E["a2b"][1]
    epad_ab = sp_ab.shape[0]
    cnt16 = _cnt_call(epad_ab, NBNP)(dp_ab)[0]
    ms4 = _passB([(sp_ab, dp_ab, dp_ab, hat4f, NAT)], NBNP, False).reshape(4, NBNP, 16)
    mx4 = _max_call(epad_ab)(sp_ab, dp_ab, hat4f)[0].reshape(4, _NRR, 16)
    out = _fc(hB[1], ms4, cnt16, mx4, p)
    return out.reshape(-1)
